# baseline (speedup 1.0000x reference)
import functools

import jax
import jax.numpy as jnp
from jax import lax
from jax.experimental import pallas as pl
from jax.experimental.pallas import tpu as pltpu
from jax.experimental.pallas import tpu_sc as plsc

_VOCAB = 100000
_EMBED = 64
_BATCH = 1024

_NC = 2
_NS = 16
_NW = _NC * _NS
_B_PER_W = _BATCH // _NW

_BM = 256
_BN = 2048
_NB = _BATCH // _BM
_NV = pl.cdiv(_VOCAB, _BN)


def _sc_gather(table, idx):
    mesh = plsc.VectorSubcoreMesh(core_axis_name="c", subcore_axis_name="s")

    @functools.partial(
        pl.kernel,
        mesh=mesh,
        out_type=jax.ShapeDtypeStruct((_BATCH, _EMBED), jnp.float32),
        scratch_types=[
            pltpu.VMEM((_B_PER_W,), jnp.int32),
            pltpu.VMEM((_B_PER_W, _EMBED), jnp.float32),
            pltpu.SemaphoreType.DMA,
        ],
        compiler_params=pltpu.CompilerParams(use_tc_tiling_on_sc=False),
    )
    def gather_kernel(table_hbm, idx_hbm, out_hbm, idx_v, rows_v, sem):
        wid = lax.axis_index("s") * _NC + lax.axis_index("c")
        base = wid * _B_PER_W
        pltpu.sync_copy(idx_hbm.at[pl.ds(base, _B_PER_W)], idx_v)
        pltpu.async_copy(table_hbm.at[idx_v], rows_v, sem).wait()
        pltpu.sync_copy(rows_v, out_hbm.at[pl.ds(base, _B_PER_W)])

    return gather_kernel(table, idx)


def _mlp_body(e_ref, wh_ref, bh_ref, wo_ref, bo_ref, out_ref):
    h = jnp.dot(e_ref[...], wh_ref[...], preferred_element_type=jnp.float32)
    h = jnp.maximum(h + bh_ref[...], 0.0)
    out = jnp.dot(h, wo_ref[...], preferred_element_type=jnp.float32)
    out_ref[...] = out + bo_ref[...]


def _tc_mlp(e, W_h, b_h, W_o, b_o):
    return pl.pallas_call(
        _mlp_body,
        grid=(_NV, _NB),
        in_specs=[
            pl.BlockSpec((_BM, _EMBED), lambda i, j: (j, 0)),
            pl.BlockSpec((_EMBED, _EMBED), lambda i, j: (0, 0)),
            pl.BlockSpec((1, _EMBED), lambda i, j: (0, 0)),
            pl.BlockSpec((_EMBED, _BN), lambda i, j: (0, i)),
            pl.BlockSpec((1, _BN), lambda i, j: (0, i)),
        ],
        out_specs=pl.BlockSpec((_BM, _BN), lambda i, j: (j, i)),
        out_shape=jax.ShapeDtypeStruct((_BATCH, _VOCAB), jnp.float32),
        compiler_params=pltpu.CompilerParams(
            dimension_semantics=("arbitrary", "arbitrary"),
        ),
    )(e, W_h, b_h, W_o, b_o)


@jax.jit
def kernel(x, table, W_h, b_h, W_o, b_o):
    idx = x.astype(jnp.int32)
    e = _sc_gather(table, idx)
    return _tc_mlp(e, W_h, jnp.reshape(b_h, (1, _EMBED)), W_o,
                   jnp.reshape(b_o, (1, _VOCAB)))


# speedup vs baseline: 1.1572x; 1.1572x over previous
import functools

import jax
import jax.numpy as jnp
from jax import lax
from jax.experimental import pallas as pl
from jax.experimental.pallas import tpu as pltpu
from jax.experimental.pallas import tpu_sc as plsc

_VOCAB = 100000
_EMBED = 64
_BATCH = 1024

_NC = 2
_NS = 16
_NW = _NC * _NS
_B_PER_W = _BATCH // _NW
_L = 16

_BN = 2048
_NV = pl.cdiv(_VOCAB, _BN)


def _sc_gather_pairs(table2, idx):
    mesh = plsc.VectorSubcoreMesh(core_axis_name="c", subcore_axis_name="s")

    @functools.partial(
        pl.kernel,
        mesh=mesh,
        out_type=jax.ShapeDtypeStruct((_BATCH, 2 * _EMBED), jnp.float32),
        scratch_types=[
            pltpu.VMEM((_B_PER_W,), jnp.int32),
            pltpu.VMEM((_B_PER_W,), jnp.int32),
            pltpu.VMEM((_B_PER_W, 2 * _EMBED), jnp.float32),
            pltpu.SemaphoreType.DMA,
        ],
    )
    def gather_kernel(table_hbm, idx_hbm, out_hbm, idx_v, idx2_v, rows_v, sem):
        wid = lax.axis_index("s") * _NC + lax.axis_index("c")
        base = wid * _B_PER_W
        pltpu.sync_copy(idx_hbm.at[pl.ds(base, _B_PER_W)], idx_v)
        for g in range(_B_PER_W // _L):
            sl = pl.ds(g * _L, _L)
            idx2_v[sl] = idx_v[sl] >> 1
        pltpu.async_copy(table_hbm.at[idx2_v], rows_v, sem).wait()
        pltpu.sync_copy(rows_v, out_hbm.at[pl.ds(base, _B_PER_W)])

    return gather_kernel(table2, idx)


def _mlp_body(e2_ref, idx_ref, whs_ref, bh_ref, wo_ref, bo_ref, out_ref, h_s):
    @pl.when(pl.program_id(0) == 0)
    def _():
        par = idx_ref[...] & 1
        half = lax.broadcasted_iota(jnp.int32, (1, 2 * _EMBED), 1) // _EMBED
        e = e2_ref[...] * (par == half).astype(jnp.float32)
        h = jnp.dot(e, whs_ref[...], preferred_element_type=jnp.float32)
        h_s[...] = jnp.maximum(h + bh_ref[...], 0.0)

    out = jnp.dot(h_s[...], wo_ref[...], preferred_element_type=jnp.float32)
    out_ref[...] = out + bo_ref[...]


def _tc_mlp(e2, idx, W_hs, b_h, W_o, b_o):
    return pl.pallas_call(
        _mlp_body,
        grid=(_NV,),
        in_specs=[
            pl.BlockSpec((_BATCH, 2 * _EMBED), lambda i: (0, 0)),
            pl.BlockSpec((_BATCH, 1), lambda i: (0, 0)),
            pl.BlockSpec((2 * _EMBED, _EMBED), lambda i: (0, 0)),
            pl.BlockSpec((1, _EMBED), lambda i: (0, 0)),
            pl.BlockSpec((_EMBED, _BN), lambda i: (0, i)),
            pl.BlockSpec((1, _BN), lambda i: (0, i)),
        ],
        out_specs=pl.BlockSpec((_BATCH, _BN), lambda i: (0, i)),
        out_shape=jax.ShapeDtypeStruct((_BATCH, _VOCAB), jnp.float32),
        scratch_shapes=[pltpu.VMEM((_BATCH, _EMBED), jnp.float32)],
        compiler_params=pltpu.CompilerParams(
            dimension_semantics=("arbitrary",),
        ),
    )(e2, idx, W_hs, b_h, W_o, b_o)


@jax.jit
def kernel(x, table, W_h, b_h, W_o, b_o):
    idx = x.astype(jnp.int32)
    table2 = jnp.reshape(table, (_VOCAB // 2, 2 * _EMBED))
    e2 = _sc_gather_pairs(table2, idx)
    W_hs = jnp.concatenate([W_h, W_h], axis=0)
    return _tc_mlp(e2, jnp.reshape(idx, (_BATCH, 1)), W_hs,
                   jnp.reshape(b_h, (1, _EMBED)), W_o,
                   jnp.reshape(b_o, (1, _VOCAB)))


# speedup vs baseline: 3.0662x; 2.6497x over previous
import functools

import jax
import jax.numpy as jnp
from jax import lax
from jax.experimental import pallas as pl
from jax.experimental.pallas import tpu as pltpu
from jax.experimental.pallas import tpu_sc as plsc

_VOCAB = 100000
_EMBED = 64
_BATCH = 1024

_NC = 2
_NS = 16
_NW = _NC * _NS
_B_PER_W = _BATCH // _NW
_L = 16

_BN = 2048
_NV = pl.cdiv(_VOCAB, _BN)


def _sc_gather_pairs(table2, idx):
    mesh = plsc.VectorSubcoreMesh(core_axis_name="c", subcore_axis_name="s")

    @functools.partial(
        pl.kernel,
        mesh=mesh,
        out_type=jax.ShapeDtypeStruct((_BATCH, 2 * _EMBED), jnp.float32),
        scratch_types=[
            pltpu.VMEM((_B_PER_W,), jnp.int32),
            pltpu.VMEM((_B_PER_W,), jnp.int32),
            pltpu.VMEM((_B_PER_W, 2 * _EMBED), jnp.float32),
            pltpu.SemaphoreType.DMA,
        ],
    )
    def gather_kernel(table_hbm, idx_hbm, out_hbm, idx_v, idx2_v, rows_v, sem):
        wid = lax.axis_index("s") * _NC + lax.axis_index("c")
        base = wid * _B_PER_W
        pltpu.sync_copy(idx_hbm.at[pl.ds(base, _B_PER_W)], idx_v)
        for g in range(_B_PER_W // _L):
            sl = pl.ds(g * _L, _L)
            idx2_v[sl] = idx_v[sl] >> 1
        pltpu.async_copy(table_hbm.at[idx2_v], rows_v, sem).wait()
        pltpu.sync_copy(rows_v, out_hbm.at[pl.ds(base, _B_PER_W)])

    return gather_kernel(table2, idx)


def _mlp_body(e2_ref, idx_ref, whs_ref, bh_ref, wo_ref, bo_ref, out_ref, h_s):
    @pl.when(pl.program_id(0) == 0)
    def _():
        par = idx_ref[...] & 1
        half = lax.broadcasted_iota(jnp.int32, (1, 2 * _EMBED), 1) // _EMBED
        e = e2_ref[...] * (par == half).astype(jnp.float32)
        h = jnp.dot(e, whs_ref[...], preferred_element_type=jnp.float32)
        h_s[...] = jnp.maximum(h + bh_ref[...], 0.0)

    out = lax.dot_general(
        wo_ref[...], h_s[...], (((0,), (1,)), ((), ())),
        preferred_element_type=jnp.float32)
    ones = jnp.ones((1, _BATCH), dtype=jnp.float32)
    out_ref[...] = out + lax.dot_general(
        bo_ref[...], ones, (((0,), (0,)), ((), ())),
        preferred_element_type=jnp.float32)


def _tc_mlp(e2, idx, W_hs, b_h, W_o, b_o):
    return pl.pallas_call(
        _mlp_body,
        grid=(_NV,),
        in_specs=[
            pl.BlockSpec((_BATCH, 2 * _EMBED), lambda i: (0, 0)),
            pl.BlockSpec((_BATCH, 1), lambda i: (0, 0)),
            pl.BlockSpec((2 * _EMBED, _EMBED), lambda i: (0, 0)),
            pl.BlockSpec((1, _EMBED), lambda i: (0, 0)),
            pl.BlockSpec((_EMBED, _BN), lambda i: (0, i)),
            pl.BlockSpec((1, _BN), lambda i: (0, i)),
        ],
        out_specs=pl.BlockSpec((_BN, _BATCH), lambda i: (i, 0)),
        out_shape=jax.ShapeDtypeStruct((_VOCAB, _BATCH), jnp.float32),
        scratch_shapes=[pltpu.VMEM((_BATCH, _EMBED), jnp.float32)],
        compiler_params=pltpu.CompilerParams(
            dimension_semantics=("arbitrary",),
        ),
    )(e2, idx, W_hs, b_h, W_o, b_o)


@jax.jit
def kernel(x, table, W_h, b_h, W_o, b_o):
    idx = x.astype(jnp.int32)
    table2 = jnp.reshape(table, (_VOCAB // 2, 2 * _EMBED))
    e2 = _sc_gather_pairs(table2, idx)
    W_hs = jnp.concatenate([W_h, W_h], axis=0)
    out_t = _tc_mlp(e2, jnp.reshape(idx, (_BATCH, 1)), W_hs,
                    jnp.reshape(b_h, (1, _EMBED)), W_o,
                    jnp.reshape(b_o, (1, _VOCAB)))
    return jnp.transpose(out_t)
